# baseline (speedup 1.0000x reference)
import jax
import jax.numpy as jnp
from jax.experimental import pallas as pl
from jax.experimental.pallas import tpu as pltpu

WINDOW = 11
EDGE = 8
TB = 8192
B, T, C = 8, 16384, 256


def _avg_kernel(xc_ref, x_hbm, o_ref, a1, a2, a3t, eh, et, sems):
    b = pl.program_id(0)
    i = pl.program_id(1)
    nt = pl.num_programs(1)
    xc = xc_ref.at[0]
    t0 = i * TB

    @pl.when(i > 0)
    def _():
        pltpu.make_async_copy(
            x_hbm.at[b, pl.ds(t0 - EDGE, EDGE), :], eh.at[0:EDGE, :], sems.at[0]
        ).start()

    @pl.when(i < nt - 1)
    def _():
        pltpu.make_async_copy(
            x_hbm.at[b, pl.ds(t0 + TB, EDGE), :], et.at[EDGE:2 * EDGE, :], sems.at[1]
        ).start()

    eh[EDGE:2 * EDGE, :] = xc[0:EDGE, :]
    et[0:EDGE, :] = xc[TB - EDGE:TB, :]
    et[2 * EDGE:3 * EDGE, :] = jnp.zeros((EDGE, C), jnp.float32)

    a1[EDGE:TB, :] = xc[0:TB - EDGE, :] + xc[1:TB - EDGE + 1, :]

    @pl.when(i > 0)
    def _():
        pltpu.make_async_copy(
            x_hbm.at[b, pl.ds(t0 - EDGE, EDGE), :], eh.at[0:EDGE, :], sems.at[0]
        ).wait()

    @pl.when(i == 0)
    def _():
        eh[0:EDGE, :] = jnp.zeros((EDGE, C), jnp.float32)

    @pl.when(i < nt - 1)
    def _():
        pltpu.make_async_copy(
            x_hbm.at[b, pl.ds(t0 + TB, EDGE), :], et.at[EDGE:2 * EDGE, :], sems.at[1]
        ).wait()

    @pl.when(i == nt - 1)
    def _():
        et[EDGE:2 * EDGE, :] = jnp.zeros((EDGE, C), jnp.float32)

    a1[0:EDGE, :] = eh[0:EDGE, :] + eh[1:EDGE + 1, :]
    a1[TB:TB + 2 * EDGE, :] = et[0:2 * EDGE, :] + et[1:2 * EDGE + 1, :]
    a2[0:TB + 16, :] = a1[0:TB + 16, :] + a1[2:TB + 18, :]
    a3t[0:TB + 8, :] = a2[0:TB + 8, :] + a2[4:TB + 12, :] + a1[8:TB + 16, :]
    o_ref[0, 0:TB - EDGE, :] = (a3t[3:TB - 5, :] + xc[5:TB - 3, :]) * (1.0 / WINDOW)
    o_ref[0, TB - EDGE:TB, :] = (a3t[TB - 5:TB + 3, :] + et[5:13, :]) * (1.0 / WINDOW)


def kernel(x):
    nt = T // TB
    grid = (B, nt)
    in_specs = [
        pl.BlockSpec((1, TB, C), lambda b, i: (b, i, 0)),
        pl.BlockSpec(memory_space=pl.ANY),
    ]
    out_spec = pl.BlockSpec((1, TB, C), lambda b, i: (b, i, 0))
    return pl.pallas_call(
        _avg_kernel,
        grid=grid,
        in_specs=in_specs,
        out_specs=out_spec,
        out_shape=jax.ShapeDtypeStruct((B, T, C), jnp.float32),
        scratch_shapes=[
            pltpu.VMEM((TB + 24, C), jnp.float32),
            pltpu.VMEM((TB + 24, C), jnp.float32),
            pltpu.VMEM((TB + 8, C), jnp.float32),
            pltpu.VMEM((2 * EDGE + 8, C), jnp.float32),
            pltpu.VMEM((3 * EDGE, C), jnp.float32),
            pltpu.SemaphoreType.DMA((2,)),
        ],
        compiler_params=pltpu.CompilerParams(
            dimension_semantics=("parallel", "arbitrary"),
            vmem_limit_bytes=64 * 1024 * 1024,
        ),
    )(x, x)


# speedup vs baseline: 1.6252x; 1.6252x over previous
import jax
import jax.numpy as jnp
from jax.experimental import pallas as pl
from jax.experimental.pallas import tpu as pltpu

WINDOW = 11
EDGE = 8
TB = 8192
B, T, C = 8, 16384, 256
assert T == 2 * TB


def _avg_kernel(xc_ref, xl_ref, xr_ref, o_ref, a1, a2, a3t, eh, et):
    i = pl.program_id(1)
    nt = pl.num_programs(1)
    xc = xc_ref.at[0]
    eh[0:EDGE, :] = jnp.where(i > 0, xl_ref[0], 0.0)
    eh[EDGE:2 * EDGE, :] = xc[0:EDGE, :]
    et[0:EDGE, :] = xc[TB - EDGE:TB, :]
    et[EDGE:2 * EDGE, :] = jnp.where(i < nt - 1, xr_ref[0], 0.0)
    et[2 * EDGE:3 * EDGE, :] = jnp.zeros((EDGE, C), jnp.float32)
    a1[0:EDGE, :] = eh[0:EDGE, :] + eh[1:EDGE + 1, :]
    a1[EDGE:TB, :] = xc[0:TB - EDGE, :] + xc[1:TB - EDGE + 1, :]
    a1[TB:TB + 2 * EDGE, :] = et[0:2 * EDGE, :] + et[1:2 * EDGE + 1, :]
    a2[0:TB + 16, :] = a1[0:TB + 16, :] + a1[2:TB + 18, :]
    a3t[0:TB + 8, :] = a2[0:TB + 8, :] + a2[4:TB + 12, :] + a1[8:TB + 16, :]
    o_ref[0, 0:TB - EDGE, :] = (a3t[3:TB - 5, :] + xc[5:TB - 3, :]) * (1.0 / WINDOW)
    o_ref[0, TB - EDGE:TB, :] = (a3t[TB - 5:TB + 3, :] + et[5:13, :]) * (1.0 / WINDOW)


def kernel(x):
    nt = T // TB
    grid = (B, nt)
    in_specs = [
        pl.BlockSpec((1, TB, C), lambda b, i: (b, i, 0)),
        pl.BlockSpec((1, EDGE, C), lambda b, i: (b, TB // EDGE - 1, 0)),
        pl.BlockSpec((1, EDGE, C), lambda b, i: (b, TB // EDGE, 0)),
    ]
    out_spec = pl.BlockSpec((1, TB, C), lambda b, i: (b, i, 0))
    return pl.pallas_call(
        _avg_kernel,
        grid=grid,
        in_specs=in_specs,
        out_specs=out_spec,
        out_shape=jax.ShapeDtypeStruct((B, T, C), jnp.float32),
        scratch_shapes=[
            pltpu.VMEM((TB + 24, C), jnp.float32),
            pltpu.VMEM((TB + 24, C), jnp.float32),
            pltpu.VMEM((TB + 8, C), jnp.float32),
            pltpu.VMEM((2 * EDGE + 8, C), jnp.float32),
            pltpu.VMEM((3 * EDGE, C), jnp.float32),
        ],
        compiler_params=pltpu.CompilerParams(
            dimension_semantics=("parallel", "arbitrary"),
            vmem_limit_bytes=64 * 1024 * 1024,
        ),
    )(x, x, x)


# speedup vs baseline: 1.6264x; 1.0008x over previous
import jax
import jax.numpy as jnp
from jax.experimental import pallas as pl
from jax.experimental.pallas import tpu as pltpu

WINDOW = 11
EDGE = 8
TB = 8192
B, T, C = 8, 16384, 256
assert T == 2 * TB


def _avg_kernel(xc_ref, xl_ref, xr_ref, o_ref, a1, a2, a3t, eh, et):
    i = pl.program_id(1)
    nt = pl.num_programs(1)
    xc = xc_ref.at[0]
    eh[0:EDGE, :] = jnp.where(i > 0, xl_ref[0], 0.0)
    eh[EDGE:2 * EDGE, :] = xc[0:EDGE, :]
    et[0:EDGE, :] = xc[TB - EDGE:TB, :]
    et[EDGE:2 * EDGE, :] = jnp.where(i < nt - 1, xr_ref[0], 0.0)
    et[2 * EDGE:3 * EDGE, :] = jnp.zeros((EDGE, C), jnp.float32)
    a1[0:EDGE, :] = eh[0:EDGE, :] + eh[1:EDGE + 1, :]
    a1[EDGE:TB, :] = xc[0:TB - EDGE, :] + xc[1:TB - EDGE + 1, :]
    a1[TB:TB + 2 * EDGE, :] = et[0:2 * EDGE, :] + et[1:2 * EDGE + 1, :]
    a2[0:TB + 16, :] = a1[0:TB + 16, :] + a1[2:TB + 18, :]
    a3t[0:TB + 8, :] = a2[0:TB + 8, :] + a2[4:TB + 12, :] + a1[8:TB + 16, :]
    o_ref[0, 0:TB - EDGE, :] = (a3t[3:TB - 5, :] + xc[5:TB - 3, :]) * (1.0 / WINDOW)
    o_ref[0, TB - EDGE:TB, :] = (a3t[TB - 5:TB + 3, :] + et[5:13, :]) * (1.0 / WINDOW)


def kernel(x):
    nt = T // TB
    grid = (B, nt)
    in_specs = [
        pl.BlockSpec((1, TB, C), lambda b, i: (b, i, 0)),
        pl.BlockSpec((1, EDGE, C), lambda b, i: (b, TB // EDGE - 1, 0)),
        pl.BlockSpec((1, EDGE, C), lambda b, i: (b, TB // EDGE, 0)),
    ]
    out_spec = pl.BlockSpec((1, TB, C), lambda b, i: (b, i, 0))
    return pl.pallas_call(
        _avg_kernel,
        grid=grid,
        in_specs=in_specs,
        out_specs=out_spec,
        out_shape=jax.ShapeDtypeStruct((B, T, C), jnp.float32),
        scratch_shapes=[
            pltpu.VMEM((TB + 24, C), jnp.float32),
            pltpu.VMEM((TB + 24, C), jnp.float32),
            pltpu.VMEM((TB + 8, C), jnp.float32),
            pltpu.VMEM((2 * EDGE + 8, C), jnp.float32),
            pltpu.VMEM((3 * EDGE, C), jnp.float32),
        ],
        compiler_params=pltpu.CompilerParams(
            dimension_semantics=("arbitrary", "arbitrary"),
            vmem_limit_bytes=64 * 1024 * 1024,
        ),
    )(x, x, x)


# speedup vs baseline: 1.6275x; 1.0007x over previous
import jax
import jax.numpy as jnp
from jax.experimental import pallas as pl
from jax.experimental.pallas import tpu as pltpu

WINDOW = 11
EDGE = 8
TB = 8192
B, T, C = 8, 16384, 256
assert T == 2 * TB


def _avg_kernel(xc_ref, xl_ref, xr_ref, o_ref, a1, eh, et):
    i = pl.program_id(1)
    nt = pl.num_programs(1)
    xc = xc_ref.at[0]
    eh[0:EDGE, :] = jnp.where(i > 0, xl_ref[0], 0.0)
    eh[EDGE:2 * EDGE, :] = xc[0:EDGE, :]
    et[0:EDGE, :] = xc[TB - EDGE:TB, :]
    et[EDGE:2 * EDGE, :] = jnp.where(i < nt - 1, xr_ref[0], 0.0)
    et[2 * EDGE:3 * EDGE, :] = jnp.zeros((EDGE, C), jnp.float32)
    a1[0:EDGE, :] = eh[0:EDGE, :] + eh[1:EDGE + 1, :]
    a1[EDGE:TB, :] = xc[0:TB - EDGE, :] + xc[1:TB - EDGE + 1, :]
    a1[TB:TB + 2 * EDGE, :] = et[0:2 * EDGE, :] + et[1:2 * EDGE + 1, :]
    o_ref[0, 0:TB - EDGE, :] = (
        ((a1[3:TB - 5, :] + a1[5:TB - 3, :]) + (a1[7:TB - 1, :] + a1[9:TB + 1, :]))
        + (a1[11:TB + 3, :] + xc[5:TB - 3, :])
    ) * (1.0 / WINDOW)
    o_ref[0, TB - EDGE:TB, :] = (
        ((a1[TB - 5:TB + 3, :] + a1[TB - 3:TB + 5, :])
         + (a1[TB - 1:TB + 7, :] + a1[TB + 1:TB + 9, :]))
        + (a1[TB + 3:TB + 11, :] + et[5:13, :])
    ) * (1.0 / WINDOW)


def kernel(x):
    nt = T // TB
    grid = (B, nt)
    in_specs = [
        pl.BlockSpec((1, TB, C), lambda b, i: (b, i, 0)),
        pl.BlockSpec((1, EDGE, C), lambda b, i: (b, TB // EDGE - 1, 0)),
        pl.BlockSpec((1, EDGE, C), lambda b, i: (b, TB // EDGE, 0)),
    ]
    out_spec = pl.BlockSpec((1, TB, C), lambda b, i: (b, i, 0))
    return pl.pallas_call(
        _avg_kernel,
        grid=grid,
        in_specs=in_specs,
        out_specs=out_spec,
        out_shape=jax.ShapeDtypeStruct((B, T, C), jnp.float32),
        scratch_shapes=[
            pltpu.VMEM((TB + 24, C), jnp.float32),
            pltpu.VMEM((2 * EDGE + 8, C), jnp.float32),
            pltpu.VMEM((3 * EDGE, C), jnp.float32),
        ],
        compiler_params=pltpu.CompilerParams(
            dimension_semantics=("arbitrary", "arbitrary"),
            vmem_limit_bytes=64 * 1024 * 1024,
        ),
    )(x, x, x)


# speedup vs baseline: 1.6286x; 1.0007x over previous
import jax
import jax.numpy as jnp
from jax.experimental import pallas as pl
from jax.experimental.pallas import tpu as pltpu

WINDOW = 11
EDGE = 8
TB = 8192
B, T, C = 8, 16384, 256
assert T == 2 * TB


def _avg_kernel(xc_ref, xl_ref, xr_ref, o_ref, a1, eh, et):
    i = pl.program_id(1)
    nt = pl.num_programs(1)
    xc = xc_ref.at[0]
    eh[0:EDGE, :] = jnp.where(i > 0, xl_ref[0], 0.0)
    eh[EDGE:2 * EDGE, :] = xc[0:EDGE, :]
    et[0:EDGE, :] = xc[TB - EDGE:TB, :]
    et[EDGE:2 * EDGE, :] = jnp.where(i < nt - 1, xr_ref[0], 0.0)
    et[2 * EDGE:3 * EDGE, :] = jnp.zeros((EDGE, C), jnp.float32)
    a1[0:EDGE, :] = eh[0:EDGE, :] + eh[1:EDGE + 1, :]
    a1[EDGE:TB, :] = xc[0:TB - EDGE, :] + xc[1:TB - EDGE + 1, :]
    a1[TB:TB + 2 * EDGE, :] = et[0:2 * EDGE, :] + et[1:2 * EDGE + 1, :]
    o_ref[0, 0:TB - EDGE, :] = (
        ((a1[3:TB - 5, :] + a1[5:TB - 3, :]) + (a1[7:TB - 1, :] + a1[9:TB + 1, :]))
        + (a1[11:TB + 3, :] + xc[5:TB - 3, :])
    ) * (1.0 / WINDOW)
    o_ref[0, TB - EDGE:TB, :] = (
        ((a1[TB - 5:TB + 3, :] + a1[TB - 3:TB + 5, :])
         + (a1[TB - 1:TB + 7, :] + a1[TB + 1:TB + 9, :]))
        + (a1[TB + 3:TB + 11, :] + et[5:13, :])
    ) * (1.0 / WINDOW)


def kernel(x):
    nt = T // TB
    grid = (B, nt)
    in_specs = [
        pl.BlockSpec((1, TB, C), lambda b, i: (b, i, 0)),
        pl.BlockSpec((1, EDGE, C), lambda b, i: (b, TB // EDGE - 1, 0)),
        pl.BlockSpec((1, EDGE, C), lambda b, i: (b, TB // EDGE, 0)),
    ]
    out_spec = pl.BlockSpec((1, TB, C), lambda b, i: (b, i, 0))
    return pl.pallas_call(
        _avg_kernel,
        grid=grid,
        in_specs=in_specs,
        out_specs=out_spec,
        out_shape=jax.ShapeDtypeStruct((B, T, C), jnp.float32),
        scratch_shapes=[
            pltpu.VMEM((TB + 24, C), jnp.float32),
            pltpu.VMEM((2 * EDGE + 8, C), jnp.float32),
            pltpu.VMEM((3 * EDGE, C), jnp.float32),
        ],
        compiler_params=pltpu.CompilerParams(
            dimension_semantics=("parallel", "arbitrary"),
            vmem_limit_bytes=64 * 1024 * 1024,
        ),
    )(x, x, x)
